# baseline (speedup 1.0000x reference)
import functools

import jax
import jax.numpy as jnp
from jax import lax
from jax.experimental import pallas as pl
from jax.experimental.pallas import tpu as pltpu
from jax.experimental.pallas import tpu_sc as plsc

VOCAB = 1000000
D = 64
BATCH = 4096
SEQ = 200
B = BATCH * SEQ
PAD_ID = 0

NC = 2
NS = 16
L = 16
NW = NC * NS
BPW = B // NW
IPG = 128
C = 512
G = C // IPG
CHUNKS = BPW // C

_mesh = plsc.VectorSubcoreMesh(core_axis_name="c", subcore_axis_name="s")


@functools.partial(
    pl.kernel,
    out_type=jax.ShapeDtypeStruct((B, D), jnp.float32),
    mesh=_mesh,
    scratch_types=[
        pltpu.VMEM((G, IPG), jnp.int32),
        pltpu.VMEM((C, D), jnp.float32),
        pltpu.SemaphoreType.DMA,
    ],
    compiler_params=pltpu.CompilerParams(use_tc_tiling_on_sc=False),
)
def _embed_lookup(ids_hbm, table_hbm, out_hbm, idx_v, rows_v, sem):
    wid = lax.axis_index("s") * NC + lax.axis_index("c")
    base = wid * BPW

    def chunk_body(k, carry):
        row0 = base + k * C

        pltpu.sync_copy(ids_hbm.at[wid * CHUNKS + k], idx_v)

        copies = [
            pltpu.async_copy(
                table_hbm.at[idx_v.at[j]],
                rows_v.at[pl.ds(j * IPG, IPG)],
                sem,
            )
            for j in range(G)
        ]
        for cp in copies:
            cp.wait()

        vs = [
            idx_v[j, pl.ds(t * L, L)]
            for j in range(G)
            for t in range(IPG // L)
        ]
        mn_vec = functools.reduce(jnp.minimum, vs)
        mn = functools.reduce(jnp.minimum, [mn_vec[i] for i in range(L)])

        @pl.when(mn == PAD_ID)
        def _fixup():
            def grp_body(g, c):
                jq = g // (IPG // L)
                tq = g % (IPG // L)
                idv = idx_v[jq, pl.ds(tq * L, L)]
                mvec = jnp.where(idv == PAD_ID, 0.0, 1.0).astype(jnp.float32)
                for rl in range(L):
                    f = mvec[rl]
                    row = g * L + rl
                    for cb in range(D // L):
                        sl = pl.ds(cb * L, L)
                        rows_v[row, sl] = rows_v[row, sl] * f
                return c

            lax.fori_loop(0, C // L, grp_body, 0)

        pltpu.sync_copy(rows_v, out_hbm.at[pl.ds(row0, C)])
        return carry

    lax.fori_loop(0, CHUNKS, chunk_body, 0)


def kernel(ids, embedding_mat):
    ids3d = ids.reshape(B // C, G, IPG)
    out = _embed_lookup(ids3d, embedding_mat)
    return out.reshape(BATCH, SEQ, D)


# speedup vs baseline: 1.0034x; 1.0034x over previous
import functools

import jax
import jax.numpy as jnp
from jax import lax
from jax.experimental import pallas as pl
from jax.experimental.pallas import tpu as pltpu
from jax.experimental.pallas import tpu_sc as plsc

VOCAB = 1000000
D = 64
BATCH = 4096
SEQ = 200
B = BATCH * SEQ
PAD_ID = 0

NC = 2
NS = 16
L = 16
NW = NC * NS
BPW = B // NW
IPG = 128
C = 512
G = C // IPG
CHUNKS = BPW // C

_mesh = plsc.VectorSubcoreMesh(core_axis_name="c", subcore_axis_name="s")


@functools.partial(
    pl.kernel,
    out_type=jax.ShapeDtypeStruct((BATCH, SEQ, D), jnp.float32),
    mesh=_mesh,
    scratch_types=[
        pltpu.VMEM((G, IPG), jnp.int32),
        pltpu.VMEM((C, D), jnp.float32),
        pltpu.SemaphoreType.DMA,
    ],
    compiler_params=pltpu.CompilerParams(use_tc_tiling_on_sc=False),
)
def _embed_lookup(ids_hbm, table_hbm, out_hbm, idx_v, rows_v, sem):
    wid = lax.axis_index("s") * NC + lax.axis_index("c")
    base = wid * BPW

    def chunk_body(k, carry):
        flat0 = base + k * C
        s = flat0 >> 12
        bblk = (flat0 & (BATCH - 1)) >> 7
        b0 = (flat0 & (BATCH - 1))

        pltpu.sync_copy(ids_hbm.at[s, pl.ds(bblk, G)], idx_v)

        copies = [
            pltpu.async_copy(
                table_hbm.at[idx_v.at[j]],
                rows_v.at[pl.ds(j * IPG, IPG)],
                sem,
            )
            for j in range(G)
        ]
        for cp in copies:
            cp.wait()

        vs = [
            idx_v[j, pl.ds(t * L, L)]
            for j in range(G)
            for t in range(IPG // L)
        ]
        mn_vec = functools.reduce(jnp.minimum, vs)
        mn = functools.reduce(jnp.minimum, [mn_vec[i] for i in range(L)])

        @pl.when(mn == PAD_ID)
        def _fixup():
            def grp_body(g, c):
                jq = g // (IPG // L)
                tq = g % (IPG // L)
                idv = idx_v[jq, pl.ds(tq * L, L)]
                mvec = jnp.where(idv == PAD_ID, 0.0, 1.0).astype(jnp.float32)
                for rl in range(L):
                    f = mvec[rl]
                    row = g * L + rl
                    for cb in range(D // L):
                        sl = pl.ds(cb * L, L)
                        rows_v[row, sl] = rows_v[row, sl] * f
                return c

            lax.fori_loop(0, C // L, grp_body, 0)

        pltpu.sync_copy(rows_v, out_hbm.at[pl.ds(b0, C), s])
        return carry

    lax.fori_loop(0, CHUNKS, chunk_body, 0)


def kernel(ids, embedding_mat):
    ids_sb = ids.T.reshape(SEQ, BATCH // IPG, IPG)
    return _embed_lookup(ids_sb, embedding_mat)
